# baseline (speedup 1.0000x reference)
import functools

import jax
import jax.numpy as jnp
from jax.experimental import pallas as pl
from jax.experimental.pallas import tpu as pltpu

_BM = 400
_BK = 2560


def _gcn_body(adj_ref, x_ref, w_ref, b_ref, out_ref, acc_ref, *, n):
    k = pl.program_id(1)
    nk = pl.num_programs(1)

    col = jax.lax.broadcasted_iota(jnp.int32, (1, _BK), 1)
    valid = (k * _BK + col) < n
    a = jnp.where(valid, adj_ref[...], 0.0)
    part = jnp.dot(
        a,
        x_ref[pl.ds(k * _BK, _BK), :],
        preferred_element_type=jnp.float32,
    )

    @pl.when(k == 0)
    def _():
        acc_ref[...] = part

    @pl.when(k != 0)
    def _():
        acc_ref[...] += part

    @pl.when(k == nk - 1)
    def _():
        out_ref[...] = (
            jnp.dot(acc_ref[...], w_ref[...], preferred_element_type=jnp.float32)
            + b_ref[...]
        )


def kernel(x, adj, W, b):
    n, din = x.shape
    dout = W.shape[1]
    nk = pl.cdiv(n, _BK)
    k_pad = nk * _BK
    b2 = b.reshape(1, dout)
    x_pad = jnp.pad(x, ((0, k_pad - n), (0, 0)))
    return pl.pallas_call(
        functools.partial(_gcn_body, n=n),
        grid=(pl.cdiv(n, _BM), nk),
        in_specs=[
            pl.BlockSpec((_BM, _BK), lambda i, k: (i, k)),
            pl.BlockSpec((k_pad, din), lambda i, k: (0, 0)),
            pl.BlockSpec((din, dout), lambda i, k: (0, 0)),
            pl.BlockSpec((1, dout), lambda i, k: (0, 0)),
        ],
        out_specs=pl.BlockSpec((_BM, dout), lambda i, k: (i, 0)),
        out_shape=jax.ShapeDtypeStruct((n, dout), jnp.float32),
        scratch_shapes=[pltpu.VMEM((_BM, dout), jnp.float32)],
        compiler_params=pltpu.CompilerParams(
            dimension_semantics=("parallel", "arbitrary"),
        ),
    )(adj, x_pad, W, b2)


# speedup vs baseline: 1.3076x; 1.3076x over previous
import jax
import jax.numpy as jnp
from jax.experimental import pallas as pl
from jax.experimental.pallas import tpu as pltpu

_BM = 400


def _gcn_body(adj_ref, x_ref, w_ref, b_ref, out_ref):
    ax = jnp.dot(adj_ref[...], x_ref[...], preferred_element_type=jnp.float32)
    out_ref[...] = (
        jnp.dot(ax, w_ref[...], preferred_element_type=jnp.float32) + b_ref[...]
    )


def kernel(x, adj, W, b):
    n, din = x.shape
    dout = W.shape[1]
    b2 = b.reshape(1, dout)
    return pl.pallas_call(
        _gcn_body,
        grid=(pl.cdiv(n, _BM),),
        in_specs=[
            pl.BlockSpec((_BM, n), lambda i: (i, 0)),
            pl.BlockSpec((n, din), lambda i: (0, 0)),
            pl.BlockSpec((din, dout), lambda i: (0, 0)),
            pl.BlockSpec((1, dout), lambda i: (0, 0)),
        ],
        out_specs=pl.BlockSpec((_BM, dout), lambda i: (i, 0)),
        out_shape=jax.ShapeDtypeStruct((n, dout), jnp.float32),
        compiler_params=pltpu.CompilerParams(
            dimension_semantics=("parallel",),
        ),
    )(adj, x, W, b2)
